# baseline (speedup 1.0000x reference)
import jax
import jax.numpy as jnp
from jax import lax
from jax.experimental import pallas as pl
from jax.experimental.pallas import tpu as pltpu, tpu_sc as plsc

N = 4096
B = 4
TOTAL = N * (N - 1) // 2
NW = 32
SPAN = TOTAL // NW
CHUNK = 17472
NCHUNK = SPAN // CHUNK
STRIDE = CHUNK + 64


def _off(i):
    return (i * ((2 * N - 1) - i)) >> 1


def _row_of(p):
    lo = jnp.int32(0)
    hi = jnp.int32(N - 1)
    for _ in range(12):
        mid = (lo + hi) >> 1
        le = _off(mid) <= p
        lo = jnp.where(le, mid, lo)
        hi = jnp.where(le, hi, mid)
    return lo


def _body(in_hbm, out_hbm, in_v, buf_v):
    wid = lax.axis_index("c") * 16 + lax.axis_index("s")
    pltpu.sync_copy(in_hbm, in_v.at[pl.ds(0, B * N)])

    def chunk_body(m, _):
        p0 = wid * SPAN + m * CHUNK
        p1 = p0 + CHUNK
        i0 = _row_of(p0)

        def row_cond(st):
            _i, off_i = st
            return off_i < p1

        def row_body(st):
            i, off_i = st
            off_next = off_i + (N - 1 - i)
            seg_start = jnp.maximum(off_i, p0)
            seg_end = jnp.minimum(off_next, p1)
            d0 = seg_start - p0
            ln = seg_end - seg_start
            ja = i + 1 + (seg_start - off_i)
            nsteps = (ln + 63) >> 6
            for b in range(B):
                ai = in_v[pl.ds(b * N + i, 16)][0]
                src = b * N + ja
                dst = b * STRIDE + d0

                def vec_body(k, _):
                    for u in range(4):
                        o = k * 64 + u * 16
                        v = in_v[pl.ds(src + o, 16)] - ai
                        buf_v[pl.ds(dst + o, 16)] = v
                    return 0

                lax.fori_loop(0, nsteps, vec_body, 0)
            return i + 1, off_next

        lax.while_loop(row_cond, row_body, (i0, _off(i0)))
        for b in range(B):
            pltpu.sync_copy(
                buf_v.at[pl.ds(b * STRIDE, CHUNK)],
                out_hbm.at[pl.ds(b * TOTAL + p0, CHUNK)],
            )
        return 0

    lax.fori_loop(0, NCHUNK, chunk_body, 0)


@jax.jit
def kernel(inputs):
    mesh = plsc.VectorSubcoreMesh(core_axis_name="c", subcore_axis_name="s")
    f = pl.kernel(
        _body,
        out_type=jax.ShapeDtypeStruct((B * TOTAL,), jnp.float32),
        mesh=mesh,
        compiler_params=pltpu.CompilerParams(needs_layout_passes=False),
        scratch_types=[
            pltpu.VMEM((B * N + 64,), jnp.float32),
            pltpu.VMEM((B * STRIDE,), jnp.float32),
        ],
    )
    return f(inputs.reshape(B * N)).reshape(B, TOTAL)


# speedup vs baseline: 7.8538x; 7.8538x over previous
import jax
import jax.numpy as jnp
from jax import lax
from jax.experimental import pallas as pl
from jax.experimental.pallas import tpu as pltpu, tpu_sc as plsc

N = 4096
B = 4
TOTAL = N * (N - 1) // 2
NT = TOTAL // 128
NW = 32
SPAN_T = 2048
CHUNK_T = 64
CHUNK = CHUNK_T * 128
TAIL_T = 48
NCHUNK = SPAN_T // CHUNK_T


def _off(i):
    return (i * ((2 * N - 1) - i)) >> 1


def _row_of(p):
    lo = jnp.int32(0)
    hi = jnp.int32(N - 1)
    for _ in range(12):
        mid = (lo + hi) >> 1
        le = _off(mid) <= p
        lo = jnp.where(le, mid, lo)
        hi = jnp.where(le, hi, mid)
    return lo


def _body(in_hbm, out_hbm, in_v, buf_v):
    wid = lax.axis_index("c") * 16 + lax.axis_index("s")
    pltpu.sync_copy(in_hbm, in_v.at[pl.ds(0, B * N)])
    lane = lax.iota(jnp.int32, 16)

    def scatter_part(b, dpos0, ln, src0, ai):
        dposv = dpos0 + lane
        mask = lane < ln
        tv = dposv >> 7
        cv = dposv & 127
        bv = jnp.full((16,), b, jnp.int32)
        val = in_v[pl.ds(src0, 16)] - ai
        plsc.store_scatter(buf_v, [tv, bv, cv], val, mask=mask)

    def fill(p0, ntiles):
        p1 = p0 + ntiles * 128
        i0 = _row_of(p0)

        def row_cond(st):
            _i, off_i = st
            return off_i < p1

        def row_body(st):
            i, off_i = st
            off_next = off_i + (N - 1 - i)
            seg_start = jnp.maximum(off_i, p0)
            seg_end = jnp.minimum(off_next, p1)
            d0 = seg_start - p0
            dend = seg_end - p0
            ja = i + 1 + (seg_start - off_i)
            dh = jnp.minimum((d0 + 15) & ~15, dend)
            df = dh + ((dend - dh) & ~15)
            nmid = (df - dh) >> 4
            for b in range(B):
                ai = in_v[pl.ds(b * N + i, 16)][0]
                src = b * N + ja - d0
                scatter_part(b, d0, dh - d0, src + d0, ai)

                def vec_body(k, _):
                    d = dh + k * 16
                    t = d >> 7
                    c = d & 127
                    buf_v[t, b, pl.ds(c, 16)] = in_v[pl.ds(src + d, 16)] - ai
                    return 0

                lax.fori_loop(0, nmid, vec_body, 0)
                scatter_part(b, df, dend - df, src + df, ai)
            return i + 1, off_next

        lax.while_loop(row_cond, row_body, (i0, _off(i0)))

    def chunk_body(m, _):
        t0 = wid * SPAN_T + m * CHUNK_T
        p0 = t0 * 128

        @pl.when((wid < NW - 1) | (m < NCHUNK - 1))
        def _full():
            fill(p0, CHUNK_T)
            pltpu.sync_copy(buf_v, out_hbm.at[pl.ds(t0, CHUNK_T)])

        @pl.when((wid == NW - 1) & (m == NCHUNK - 1))
        def _tail():
            fill(p0, TAIL_T)
            pltpu.sync_copy(
                buf_v.at[pl.ds(0, TAIL_T)],
                out_hbm.at[pl.ds(t0, TAIL_T)],
            )

        return 0

    lax.fori_loop(0, NCHUNK, chunk_body, 0)


@jax.jit
def kernel(inputs):
    mesh = plsc.VectorSubcoreMesh(core_axis_name="c", subcore_axis_name="s")
    f = pl.kernel(
        _body,
        out_type=jax.ShapeDtypeStruct((NT, B, 128), jnp.float32),
        mesh=mesh,
        compiler_params=pltpu.CompilerParams(needs_layout_passes=False),
        scratch_types=[
            pltpu.VMEM((B * N + 16,), jnp.float32),
            pltpu.VMEM((CHUNK_T, B, 128), jnp.float32),
        ],
    )
    out3 = f(inputs.reshape(B * N))
    return out3.transpose(1, 0, 2).reshape(B, TOTAL)
